# Initial kernel scaffold; baseline (speedup 1.0000x reference)
import functools

import jax
import jax.numpy as jnp
import numpy as np
from jax import lax
from jax.experimental import pallas as pl
from jax.experimental.pallas import tpu as pltpu
from jax.experimental.pallas import tpu_sc as plsc

_N = 10000
_E = 160000
_F_IN = 22
_F_E = 4
_H = 32
_EPS = 1e-5
_D = 32

_NC = 2
_NS = 16
_NW = _NC * _NS
_CHUNK = 125
_CPW = _E // (_NW * _CHUNK)
_EPW = _CHUNK * _CPW
_RPS = _N // _NS

_BLK = 1000
_NBLK = _E // _BLK


def _expand_mats(in_ch):
    ch = in_ch * _H
    r = np.zeros((_D, ch), np.float32)
    s = np.zeros((ch, _H), np.float32)
    for i in range(in_ch):
        r[i, i * _H:(i + 1) * _H] = 1.0
        s[i * _H:(i + 1) * _H, :] = np.eye(_H, dtype=np.float32)
    return jnp.asarray(r), jnp.asarray(s)


_R1, _S1 = _expand_mats(_F_IN)
_R2, _S2 = _expand_mats(_H)

_sc_mesh = plsc.VectorSubcoreMesh(core_axis_name="c", subcore_axis_name="s",
                                  num_cores=_NC, num_subcores=_NS)


def _gather_body(table_hbm, idx_hbm, out_hbm, idx_v, rows_v, sem):
    wid = lax.axis_index("s") * _NC + lax.axis_index("c")
    pltpu.sync_copy(idx_hbm.at[wid], idx_v)
    base = wid * _EPW

    def step(j, carry):
        pltpu.async_copy(table_hbm.at[idx_v.at[j]], rows_v, sem).wait()
        pltpu.sync_copy(rows_v, out_hbm.at[pl.ds(base + j * _CHUNK, _CHUNK)])
        return carry

    lax.fori_loop(0, _CPW, step, 0)


_gather = pl.kernel(
    _gather_body,
    out_type=jax.ShapeDtypeStruct((_E, _D), jnp.float32),
    mesh=_sc_mesh,
    scratch_types=[
        pltpu.VMEM((_CPW, _CHUNK), jnp.int32),
        pltpu.VMEM((_CHUNK, _D), jnp.float32),
        pltpu.SemaphoreType.DMA,
    ],
)


def _scatter_body(msg_hbm, idx_hbm, zeros_hbm, out_hbm, idx_v, rows_v, acc_sh,
                  sem):
    cid = lax.axis_index("c")
    sid = lax.axis_index("s")
    wid = sid * _NC + cid
    pltpu.sync_copy(zeros_hbm.at[pl.ds(sid * _RPS, _RPS)],
                    acc_sh.at[pl.ds(sid * _RPS, _RPS)])
    plsc.subcore_barrier()
    pltpu.sync_copy(idx_hbm.at[wid], idx_v)
    base = wid * _EPW

    def step(j, carry):
        pltpu.sync_copy(msg_hbm.at[pl.ds(base + j * _CHUNK, _CHUNK)], rows_v)
        pltpu.sync_copy(rows_v, acc_sh.at[idx_v.at[j]], add=True)
        return carry

    lax.fori_loop(0, _CPW, step, 0)
    plsc.subcore_barrier()
    pltpu.sync_copy(acc_sh.at[pl.ds(sid * _RPS, _RPS)],
                    out_hbm.at[cid, pl.ds(sid * _RPS, _RPS)])


_scatter = pl.kernel(
    _scatter_body,
    out_type=jax.ShapeDtypeStruct((_NC, _N, _D), jnp.float32),
    mesh=_sc_mesh,
    scratch_types=[
        pltpu.VMEM((_CPW, _CHUNK), jnp.int32),
        pltpu.VMEM((_CHUNK, _D), jnp.float32),
        pltpu.VMEM_SHARED((_N, _D), jnp.float32),
        pltpu.SemaphoreType.DMA,
    ],
)


def _fused_body(ea_ref, xg_ref, w_ref, b_ref, r_ref, s_ref, out_ref):
    z = jnp.dot(ea_ref[...], w_ref[...], preferred_element_type=jnp.float32)
    z = jnp.maximum(z + b_ref[...], 0.0)
    xr = jnp.dot(xg_ref[...], r_ref[...], preferred_element_type=jnp.float32)
    out_ref[...] = jnp.dot(z * xr, s_ref[...],
                           preferred_element_type=jnp.float32)


def _fused_msgs(ea, xg, nn_w, nn_b, r, s):
    ch = nn_w.shape[1]
    return pl.pallas_call(
        _fused_body,
        grid=(_NBLK,),
        in_specs=[
            pl.BlockSpec((_BLK, _F_E), lambda i: (i, 0)),
            pl.BlockSpec((_BLK, _D), lambda i: (i, 0)),
            pl.BlockSpec((_F_E, ch), lambda i: (0, 0)),
            pl.BlockSpec((1, ch), lambda i: (0, 0)),
            pl.BlockSpec((_D, ch), lambda i: (0, 0)),
            pl.BlockSpec((ch, _H), lambda i: (0, 0)),
        ],
        out_specs=pl.BlockSpec((_BLK, _H), lambda i: (i, 0)),
        out_shape=jax.ShapeDtypeStruct((_E, _H), jnp.float32),
    )(ea, xg, nn_w, nn_b.reshape(1, ch), r, s)


def _bn1_body(aggp_ref, x_ref, w_ref, b_ref, g_ref, bt_ref, out_ref):
    pre = (aggp_ref[0] + aggp_ref[1]
           + jnp.dot(x_ref[...], w_ref[...],
                     preferred_element_type=jnp.float32) + b_ref[...])
    m = jnp.mean(pre, axis=0, keepdims=True)
    v = jnp.mean((pre - m) * (pre - m), axis=0, keepdims=True)
    h = (pre - m) * lax.rsqrt(v + _EPS) * g_ref[...] + bt_ref[...]
    out_ref[...] = jnp.maximum(h, 0.0)


def _bn2_body(aggp_ref, h_ref, w_ref, b_ref, g_ref, bt_ref, fcw_ref, fcb_ref,
              out_ref):
    pre = (aggp_ref[0] + aggp_ref[1]
           + jnp.dot(h_ref[...], w_ref[...],
                     preferred_element_type=jnp.float32) + b_ref[...])
    m = jnp.mean(pre, axis=0, keepdims=True)
    v = jnp.mean((pre - m) * (pre - m), axis=0, keepdims=True)
    h2 = (pre - m) * lax.rsqrt(v + _EPS) * g_ref[...] + bt_ref[...]
    h2 = jnp.maximum(h2, 0.0)
    logit = jnp.dot(h2, fcw_ref[...],
                    preferred_element_type=jnp.float32) + fcb_ref[...]
    out_ref[...] = 1.0 / (1.0 + jnp.exp(-logit))


def _bn1(aggp, x, w, b, g, bt):
    return pl.pallas_call(
        _bn1_body,
        out_shape=jax.ShapeDtypeStruct((_N, _H), jnp.float32),
    )(aggp, x, w, b.reshape(1, _H), g.reshape(1, _H), bt.reshape(1, _H))


def _bn2fc(aggp, h, w, b, g, bt, fcw, fcb):
    return pl.pallas_call(
        _bn2_body,
        out_shape=jax.ShapeDtypeStruct((_N, 1), jnp.float32),
    )(aggp, h, w, b.reshape(1, _H), g.reshape(1, _H), bt.reshape(1, _H),
      fcw, fcb.reshape(1, 1))


def kernel(x, edge_index, edge_attr, en1_W, en1_b, root1_W, bias1, en2_W,
           en2_b, root2_W, bias2, bn1_gamma, bn1_beta, bn2_gamma, bn2_beta,
           fc_W, fc_b):
    src = edge_index[0].reshape(_NW, _CPW, _CHUNK)
    dst = edge_index[1].reshape(_NW, _CPW, _CHUNK)
    xpad = jnp.pad(x, ((0, 0), (0, _D - _F_IN)))
    zeros = jnp.zeros((_N, _D), jnp.float32)

    xg = _gather(xpad, src)
    msg1 = _fused_msgs(edge_attr, xg, en1_W, en1_b, _R1, _S1)
    aggp1 = _scatter(msg1, dst, zeros)
    h = _bn1(aggp1, x, root1_W, bias1, bn1_gamma, bn1_beta)

    hg = _gather(h, src)
    msg2 = _fused_msgs(edge_attr, hg, en2_W, en2_b, _R2, _S2)
    aggp2 = _scatter(msg2, dst, zeros)
    return _bn2fc(aggp2, h, root2_W, bias2, bn2_gamma, bn2_beta, fc_W, fc_b)


# speedup vs baseline: 2.0385x; 2.0385x over previous
import functools

import jax
import jax.numpy as jnp
import numpy as np
from jax import lax
from jax.experimental import pallas as pl
from jax.experimental.pallas import tpu as pltpu
from jax.experimental.pallas import tpu_sc as plsc

_N = 10000
_E = 160000
_F_IN = 22
_F_E = 4
_H = 32
_EPS = 1e-5
_D = 32
_NPAD = 10240

_NC = 2
_NS = 16
_NW = _NC * _NS
_CHUNK = 128
_CPW = 40
_EPW = _CHUNK * _CPW
_EP = _EPW * _NW
_RPS = _NPAD // _NS

_BLK = 1024
_NBLK = _EP // _BLK


def _expand_mats(in_ch):
    ch = in_ch * _H
    r = np.zeros((_D, ch), np.float32)
    s = np.zeros((ch, _H), np.float32)
    for i in range(in_ch):
        r[i, i * _H:(i + 1) * _H] = 1.0
        s[i * _H:(i + 1) * _H, :] = np.eye(_H, dtype=np.float32)
    return r, s


_R1, _S1 = _expand_mats(_F_IN)
_R2, _S2 = _expand_mats(_H)


@functools.cache
def _sc_mesh():
    return plsc.VectorSubcoreMesh(core_axis_name="c", subcore_axis_name="s",
                                  num_cores=_NC, num_subcores=_NS)


def _gather_body(table_hbm, idx_hbm, out_hbm, idx_v, rows_v, sem):
    wid = lax.axis_index("s") * _NC + lax.axis_index("c")
    pltpu.sync_copy(idx_hbm.at[wid], idx_v)
    base = wid * _EPW

    def step(j, carry):
        pltpu.async_copy(table_hbm.at[idx_v.at[j]], rows_v, sem).wait()
        pltpu.sync_copy(rows_v, out_hbm.at[pl.ds(base + j * _CHUNK, _CHUNK)])
        return carry

    lax.fori_loop(0, _CPW, step, 0)


@functools.cache
def _gather_kernel():
    return pl.kernel(
        _gather_body,
        out_type=jax.ShapeDtypeStruct((_EP, _D), jnp.float32),
        mesh=_sc_mesh(),
        compiler_params=pltpu.CompilerParams(use_tc_tiling_on_sc=False),
        scratch_types=[
            pltpu.VMEM((_CPW, _CHUNK), jnp.int32),
            pltpu.VMEM((_CHUNK, _D), jnp.float32),
            pltpu.SemaphoreType.DMA,
        ],
    )


def _gather(table, idx):
    return _gather_kernel()(table, idx)


def _scatter_body(msg_hbm, idx_hbm, zeros_hbm, out_hbm, idx_v, rows_v, acc_sh,
                  sem):
    cid = lax.axis_index("c")
    sid = lax.axis_index("s")
    wid = sid * _NC + cid
    pltpu.sync_copy(zeros_hbm.at[pl.ds(sid * _RPS, _RPS)],
                    acc_sh.at[pl.ds(sid * _RPS, _RPS)])
    plsc.subcore_barrier()
    pltpu.sync_copy(idx_hbm.at[wid], idx_v)
    base = wid * _EPW

    def step(j, carry):
        pltpu.sync_copy(msg_hbm.at[pl.ds(base + j * _CHUNK, _CHUNK)], rows_v)
        pltpu.sync_copy(rows_v, acc_sh.at[idx_v.at[j]], add=True)
        return carry

    lax.fori_loop(0, _CPW, step, 0)
    plsc.subcore_barrier()
    pltpu.sync_copy(acc_sh.at[pl.ds(sid * _RPS, _RPS)],
                    out_hbm.at[cid, pl.ds(sid * _RPS, _RPS)])


@functools.cache
def _scatter_kernel():
    return pl.kernel(
        _scatter_body,
        out_type=jax.ShapeDtypeStruct((_NC, _NPAD, _D), jnp.float32),
        mesh=_sc_mesh(),
        compiler_params=pltpu.CompilerParams(use_tc_tiling_on_sc=False),
        scratch_types=[
            pltpu.VMEM((_CPW, _CHUNK), jnp.int32),
            pltpu.VMEM((_CHUNK, _D), jnp.float32),
            pltpu.VMEM_SHARED((_NPAD, _D), jnp.float32),
            pltpu.SemaphoreType.DMA,
        ],
    )


def _scatter(msg, idx, zeros):
    return _scatter_kernel()(msg, idx, zeros)


def _fused_body(ea_ref, xg_ref, w_ref, b_ref, r_ref, s_ref, out_ref):
    z = jnp.dot(ea_ref[...], w_ref[...], preferred_element_type=jnp.float32)
    z = jnp.maximum(z + b_ref[...], 0.0)
    xr = jnp.dot(xg_ref[...], r_ref[...], preferred_element_type=jnp.float32)
    out_ref[...] = jnp.dot(z * xr, s_ref[...],
                           preferred_element_type=jnp.float32)


def _fused_msgs(ea, xg, nn_w, nn_b, r, s):
    ch = nn_w.shape[1]
    return pl.pallas_call(
        _fused_body,
        grid=(_NBLK,),
        in_specs=[
            pl.BlockSpec((_BLK, _F_E), lambda i: (i, 0)),
            pl.BlockSpec((_BLK, _D), lambda i: (i, 0)),
            pl.BlockSpec((_F_E, ch), lambda i: (0, 0)),
            pl.BlockSpec((1, ch), lambda i: (0, 0)),
            pl.BlockSpec((_D, ch), lambda i: (0, 0)),
            pl.BlockSpec((ch, _H), lambda i: (0, 0)),
        ],
        out_specs=pl.BlockSpec((_BLK, _H), lambda i: (i, 0)),
        out_shape=jax.ShapeDtypeStruct((_EP, _H), jnp.float32),
    )(ea, xg, nn_w, nn_b.reshape(1, ch), r, s)


def _bn1_body(aggp_ref, x_ref, w_ref, b_ref, g_ref, bt_ref, out_ref):
    agg = (aggp_ref[0] + aggp_ref[1])[:_N]
    pre = (agg + jnp.dot(x_ref[...], w_ref[...],
                         preferred_element_type=jnp.float32) + b_ref[...])
    m = jnp.mean(pre, axis=0, keepdims=True)
    v = jnp.mean((pre - m) * (pre - m), axis=0, keepdims=True)
    h = (pre - m) * lax.rsqrt(v + _EPS) * g_ref[...] + bt_ref[...]
    out_ref[...] = jnp.maximum(h, 0.0)


def _bn2_body(aggp_ref, h_ref, w_ref, b_ref, g_ref, bt_ref, fcw_ref, fcb_ref,
              out_ref):
    agg = (aggp_ref[0] + aggp_ref[1])[:_N]
    pre = (agg + jnp.dot(h_ref[...], w_ref[...],
                         preferred_element_type=jnp.float32) + b_ref[...])
    m = jnp.mean(pre, axis=0, keepdims=True)
    v = jnp.mean((pre - m) * (pre - m), axis=0, keepdims=True)
    h2 = (pre - m) * lax.rsqrt(v + _EPS) * g_ref[...] + bt_ref[...]
    h2 = jnp.maximum(h2, 0.0)
    logit = jnp.dot(h2, fcw_ref[...],
                    preferred_element_type=jnp.float32) + fcb_ref[...]
    out_ref[...] = 1.0 / (1.0 + jnp.exp(-logit))


def _bn1(aggp, x, w, b, g, bt):
    return pl.pallas_call(
        _bn1_body,
        out_shape=jax.ShapeDtypeStruct((_N, _H), jnp.float32),
    )(aggp, x, w, b.reshape(1, _H), g.reshape(1, _H), bt.reshape(1, _H))


def _bn2fc(aggp, h, w, b, g, bt, fcw, fcb):
    return pl.pallas_call(
        _bn2_body,
        out_shape=jax.ShapeDtypeStruct((_N, 1), jnp.float32),
    )(aggp, h, w, b.reshape(1, _H), g.reshape(1, _H), bt.reshape(1, _H),
      fcw, fcb.reshape(1, 1))


def kernel(x, edge_index, edge_attr, en1_W, en1_b, root1_W, bias1, en2_W,
           en2_b, root2_W, bias2, bn1_gamma, bn1_beta, bn2_gamma, bn2_beta,
           fc_W, fc_b):
    pad_e = _EP - _E
    src = jnp.pad(edge_index[0], (0, pad_e)).reshape(_NW, _CPW, _CHUNK)
    dst = jnp.pad(edge_index[1], (0, pad_e),
                  constant_values=_N).reshape(_NW, _CPW, _CHUNK)
    ea = jnp.pad(edge_attr, ((0, pad_e), (0, 0)))
    xpad = jnp.pad(x, ((0, 0), (0, _D - _F_IN)))
    zeros = jnp.zeros((_NPAD, _D), jnp.float32)

    xg = _gather(xpad, src)
    msg1 = _fused_msgs(ea, xg, en1_W, en1_b, _R1, _S1)
    aggp1 = _scatter(msg1, dst, zeros)
    h = _bn1(aggp1, x, root1_W, bias1, bn1_gamma, bn1_beta)

    hg = _gather(h, src)
    msg2 = _fused_msgs(ea, hg, en2_W, en2_b, _R2, _S2)
    aggp2 = _scatter(msg2, dst, zeros)
    return _bn2fc(aggp2, h, root2_W, bias2, bn2_gamma, bn2_beta, fc_W, fc_b)


# speedup vs baseline: 2.1427x; 1.0511x over previous
import functools

import jax
import jax.numpy as jnp
import numpy as np
from jax import lax
from jax.experimental import pallas as pl
from jax.experimental.pallas import tpu as pltpu
from jax.experimental.pallas import tpu_sc as plsc

_N = 10000
_E = 160000
_F_IN = 22
_F_E = 4
_H = 32
_EPS = 1e-5
_D = 32
_NPAD = 10240

_NC = 2
_NS = 16
_NW = _NC * _NS
_CHUNK = 128
_CPW = 40
_EPW = _CHUNK * _CPW
_EP = _EPW * _NW
_RPS = _NPAD // _NS

_BLK = 1024
_NBLK = _EP // _BLK


def _expand_mats(in_ch):
    ch = in_ch * _H
    r = np.zeros((_D, ch), np.float32)
    s = np.zeros((ch, _H), np.float32)
    for i in range(in_ch):
        r[i, i * _H:(i + 1) * _H] = 1.0
        s[i * _H:(i + 1) * _H, :] = np.eye(_H, dtype=np.float32)
    return r, s


_R1, _S1 = _expand_mats(_F_IN)
_R2, _S2 = _expand_mats(_H)


@functools.cache
def _sc_mesh():
    return plsc.VectorSubcoreMesh(core_axis_name="c", subcore_axis_name="s",
                                  num_cores=_NC, num_subcores=_NS)


def _gather_body(table_hbm, idx_hbm, out_hbm, idx_v, rows0, rows1, sem0,
                 sem1):
    wid = lax.axis_index("s") * _NC + lax.axis_index("c")
    pltpu.sync_copy(idx_hbm.at[wid], idx_v)
    base = wid * _EPW
    bufs = ((rows0, sem0), (rows1, sem1))
    pltpu.async_copy(table_hbm.at[idx_v.at[0]], rows0, sem0)
    pltpu.async_copy(table_hbm.at[idx_v.at[1]], rows1, sem1)

    def pair(k, carry):
        for b in range(2):
            rows, sem = bufs[b]
            g = 2 * k + b
            pltpu.make_async_copy(table_hbm.at[idx_v.at[g]], rows, sem).wait()
            pltpu.sync_copy(rows, out_hbm.at[pl.ds(base + g * _CHUNK, _CHUNK)])

            @pl.when(g + 2 < _CPW)
            def _(rows=rows, sem=sem, g=g):
                pltpu.async_copy(table_hbm.at[idx_v.at[g + 2]], rows, sem)

        return carry

    lax.fori_loop(0, _CPW // 2, pair, 0)


@functools.cache
def _gather_kernel():
    return pl.kernel(
        _gather_body,
        out_type=jax.ShapeDtypeStruct((_EP, _D), jnp.float32),
        mesh=_sc_mesh(),
        compiler_params=pltpu.CompilerParams(use_tc_tiling_on_sc=False),
        scratch_types=[
            pltpu.VMEM((_CPW, _CHUNK), jnp.int32),
            pltpu.VMEM((_CHUNK, _D), jnp.float32),
            pltpu.VMEM((_CHUNK, _D), jnp.float32),
            pltpu.SemaphoreType.DMA,
            pltpu.SemaphoreType.DMA,
        ],
    )


def _gather(table, idx):
    return _gather_kernel()(table, idx)


def _scatter_body(msg_hbm, idx_hbm, zeros_hbm, out_hbm, idx_v, rows0, rows1,
                  acc_sh, sem0, sem1):
    cid = lax.axis_index("c")
    sid = lax.axis_index("s")
    wid = sid * _NC + cid
    pltpu.sync_copy(zeros_hbm.at[pl.ds(sid * _RPS, _RPS)],
                    acc_sh.at[pl.ds(sid * _RPS, _RPS)])
    plsc.subcore_barrier()
    pltpu.sync_copy(idx_hbm.at[wid], idx_v)
    base = wid * _EPW
    bufs = ((rows0, sem0), (rows1, sem1))
    pltpu.async_copy(msg_hbm.at[pl.ds(base, _CHUNK)], rows0, sem0)
    pltpu.async_copy(msg_hbm.at[pl.ds(base + _CHUNK, _CHUNK)], rows1, sem1)

    def pair(k, carry):
        for b in range(2):
            rows, sem = bufs[b]
            g = 2 * k + b
            pltpu.make_async_copy(
                msg_hbm.at[pl.ds(base + g * _CHUNK, _CHUNK)], rows,
                sem).wait()
            pltpu.sync_copy(rows, acc_sh.at[idx_v.at[g]], add=True)

            @pl.when(g + 2 < _CPW)
            def _(rows=rows, sem=sem, g=g):
                pltpu.async_copy(
                    msg_hbm.at[pl.ds(base + (g + 2) * _CHUNK, _CHUNK)], rows,
                    sem)

        return carry

    lax.fori_loop(0, _CPW // 2, pair, 0)
    plsc.subcore_barrier()
    pltpu.sync_copy(acc_sh.at[pl.ds(sid * _RPS, _RPS)],
                    out_hbm.at[cid, pl.ds(sid * _RPS, _RPS)])


@functools.cache
def _scatter_kernel():
    return pl.kernel(
        _scatter_body,
        out_type=jax.ShapeDtypeStruct((_NC, _NPAD, _D), jnp.float32),
        mesh=_sc_mesh(),
        compiler_params=pltpu.CompilerParams(use_tc_tiling_on_sc=False),
        scratch_types=[
            pltpu.VMEM((_CPW, _CHUNK), jnp.int32),
            pltpu.VMEM((_CHUNK, _D), jnp.float32),
            pltpu.VMEM((_CHUNK, _D), jnp.float32),
            pltpu.VMEM_SHARED((_NPAD, _D), jnp.float32),
            pltpu.SemaphoreType.DMA,
            pltpu.SemaphoreType.DMA,
        ],
    )


def _scatter(msg, idx, zeros):
    return _scatter_kernel()(msg, idx, zeros)


def _fused_body(ea_ref, xg_ref, w_ref, b_ref, r_ref, s_ref, out_ref):
    z = jnp.dot(ea_ref[...], w_ref[...], preferred_element_type=jnp.float32)
    z = jnp.maximum(z + b_ref[...], 0.0)
    xr = jnp.dot(xg_ref[...], r_ref[...], preferred_element_type=jnp.float32)
    out_ref[...] = jnp.dot(z * xr, s_ref[...],
                           preferred_element_type=jnp.float32)


def _fused_msgs(ea, xg, nn_w, nn_b, r, s):
    ch = nn_w.shape[1]
    return pl.pallas_call(
        _fused_body,
        grid=(_NBLK,),
        in_specs=[
            pl.BlockSpec((_BLK, _F_E), lambda i: (i, 0)),
            pl.BlockSpec((_BLK, _D), lambda i: (i, 0)),
            pl.BlockSpec((_F_E, ch), lambda i: (0, 0)),
            pl.BlockSpec((1, ch), lambda i: (0, 0)),
            pl.BlockSpec((_D, ch), lambda i: (0, 0)),
            pl.BlockSpec((ch, _H), lambda i: (0, 0)),
        ],
        out_specs=pl.BlockSpec((_BLK, _H), lambda i: (i, 0)),
        out_shape=jax.ShapeDtypeStruct((_EP, _H), jnp.float32),
    )(ea, xg, nn_w, nn_b.reshape(1, ch), r, s)


def _bn1_body(aggp_ref, x_ref, w_ref, b_ref, g_ref, bt_ref, out_ref):
    agg = (aggp_ref[0] + aggp_ref[1])[:_N]
    pre = (agg + jnp.dot(x_ref[...], w_ref[...],
                         preferred_element_type=jnp.float32) + b_ref[...])
    m = jnp.mean(pre, axis=0, keepdims=True)
    v = jnp.mean((pre - m) * (pre - m), axis=0, keepdims=True)
    h = (pre - m) * lax.rsqrt(v + _EPS) * g_ref[...] + bt_ref[...]
    out_ref[...] = jnp.maximum(h, 0.0)


def _bn2_body(aggp_ref, h_ref, w_ref, b_ref, g_ref, bt_ref, fcw_ref, fcb_ref,
              out_ref):
    agg = (aggp_ref[0] + aggp_ref[1])[:_N]
    pre = (agg + jnp.dot(h_ref[...], w_ref[...],
                         preferred_element_type=jnp.float32) + b_ref[...])
    m = jnp.mean(pre, axis=0, keepdims=True)
    v = jnp.mean((pre - m) * (pre - m), axis=0, keepdims=True)
    h2 = (pre - m) * lax.rsqrt(v + _EPS) * g_ref[...] + bt_ref[...]
    h2 = jnp.maximum(h2, 0.0)
    logit = jnp.dot(h2, fcw_ref[...],
                    preferred_element_type=jnp.float32) + fcb_ref[...]
    out_ref[...] = 1.0 / (1.0 + jnp.exp(-logit))


def _bn1(aggp, x, w, b, g, bt):
    return pl.pallas_call(
        _bn1_body,
        out_shape=jax.ShapeDtypeStruct((_N, _H), jnp.float32),
    )(aggp, x, w, b.reshape(1, _H), g.reshape(1, _H), bt.reshape(1, _H))


def _bn2fc(aggp, h, w, b, g, bt, fcw, fcb):
    return pl.pallas_call(
        _bn2_body,
        out_shape=jax.ShapeDtypeStruct((_N, 1), jnp.float32),
    )(aggp, h, w, b.reshape(1, _H), g.reshape(1, _H), bt.reshape(1, _H),
      fcw, fcb.reshape(1, 1))


def kernel(x, edge_index, edge_attr, en1_W, en1_b, root1_W, bias1, en2_W,
           en2_b, root2_W, bias2, bn1_gamma, bn1_beta, bn2_gamma, bn2_beta,
           fc_W, fc_b):
    pad_e = _EP - _E
    src = jnp.pad(edge_index[0], (0, pad_e)).reshape(_NW, _CPW, _CHUNK)
    dst = jnp.pad(edge_index[1], (0, pad_e),
                  constant_values=_N).reshape(_NW, _CPW, _CHUNK)
    ea = jnp.pad(edge_attr, ((0, pad_e), (0, 0)))
    xpad = jnp.pad(x, ((0, 0), (0, _D - _F_IN)))
    zeros = jnp.zeros((_NPAD, _D), jnp.float32)

    xg = _gather(xpad, src)
    msg1 = _fused_msgs(ea, xg, en1_W, en1_b, _R1, _S1)
    aggp1 = _scatter(msg1, dst, zeros)
    h = _bn1(aggp1, x, root1_W, bias1, bn1_gamma, bn1_beta)

    hg = _gather(h, src)
    msg2 = _fused_msgs(ea, hg, en2_W, en2_b, _R2, _S2)
    aggp2 = _scatter(msg2, dst, zeros)
    return _bn2fc(aggp2, h, root2_W, bias2, bn2_gamma, bn2_beta, fc_W, fc_b)
